# Initial kernel scaffold; baseline (speedup 1.0000x reference)
import jax
import jax.numpy as jnp
from jax.experimental import pallas as pl


def kernel(features, edge_index_rel0, edge_index_rel1, edge_index_rel2, W_rel0, W_rel1, W_rel2, W_self):
    raise NotImplementedError("write your pallas kernel here")


# speedup vs baseline: 5.4764x; 5.4764x over previous
import jax
import jax.numpy as jnp
from jax import lax
from jax.experimental import pallas as pl
from jax.experimental.pallas import tpu as pltpu
from jax.experimental.pallas import tpu_sc as plsc

N = 10000
E = 160000
D = 128

NC = 2
NS = 16
L = 16

EDGES_PER_CORE = E // NC
EDGES_PER_TILE = EDGES_PER_CORE // NS
CHUNK = 128
NFULL = EDGES_PER_TILE // CHUNK
TAIL = EDGES_PER_TILE - NFULL * CHUNK

N_PAD = 10240
ROWS_PER_TILE = N_PAD // NS
ZROWS = 80


def _sc_body(feat_hbm, e0_hbm, e1_hbm, e2_hbm, out_hbm,
             accum, idx_src, idx_dst, rows, idx_src_t, idx_dst_t, rows_t,
             zbuf, gsem):
    c = lax.axis_index("c")
    s = lax.axis_index("s")
    row0 = s * ROWS_PER_TILE

    z16 = jnp.zeros((L,), jnp.float32)

    def zfill(i, carry):
        for k in range(D // L):
            zbuf[i, pl.ds(k * L, L)] = z16
        return carry
    lax.fori_loop(0, ZROWS, zfill, 0)

    def zero_own_rows():
        def zcopy(k, carry):
            pltpu.sync_copy(zbuf, accum.at[pl.ds(row0 + k * ZROWS, ZROWS)])
            return carry
        lax.fori_loop(0, ROWS_PER_TILE // ZROWS, zcopy, 0)

    zero_own_rows()
    plsc.subcore_barrier()

    ebase = c * EDGES_PER_CORE + s * EDGES_PER_TILE
    for r, e_hbm in enumerate((e0_hbm, e1_hbm, e2_hbm)):
        def chunk_body(j, carry, e_hbm=e_hbm):
            base = ebase + j * CHUNK
            pltpu.sync_copy(e_hbm.at[pl.ds(base, CHUNK)], idx_src)
            pltpu.sync_copy(e_hbm.at[pl.ds(E + base, CHUNK)], idx_dst)
            pltpu.async_copy(feat_hbm.at[idx_src], rows, gsem).wait()
            pltpu.sync_copy(rows, accum.at[idx_dst], add=True)
            return carry
        lax.fori_loop(0, NFULL, chunk_body, 0)

        tbase = ebase + NFULL * CHUNK
        pltpu.sync_copy(e_hbm.at[pl.ds(tbase, TAIL)], idx_src_t)
        pltpu.sync_copy(e_hbm.at[pl.ds(E + tbase, TAIL)], idx_dst_t)
        pltpu.async_copy(feat_hbm.at[idx_src_t], rows_t, gsem).wait()
        pltpu.sync_copy(rows_t, accum.at[idx_dst_t], add=True)

        plsc.subcore_barrier()
        oc = c * 3 + r
        sl = pl.ds(row0, ROWS_PER_TILE)
        pltpu.sync_copy(accum.at[sl], out_hbm.at[oc, sl])
        if r < 2:
            zero_own_rows()
        plsc.subcore_barrier()


def _sc_accumulate(features, ei0, ei1, ei2):
    mesh = plsc.VectorSubcoreMesh(core_axis_name="c", subcore_axis_name="s",
                                  num_cores=NC, num_subcores=NS)
    return pl.kernel(
        _sc_body,
        out_type=jax.ShapeDtypeStruct((2 * 3, N_PAD, D), jnp.float32),
        mesh=mesh,
        scratch_types=[
            pltpu.VMEM_SHARED((N_PAD, D), jnp.float32),
            pltpu.VMEM((CHUNK,), jnp.int32),
            pltpu.VMEM((CHUNK,), jnp.int32),
            pltpu.VMEM((CHUNK, D), jnp.float32),
            pltpu.VMEM((TAIL,), jnp.int32),
            pltpu.VMEM((TAIL,), jnp.int32),
            pltpu.VMEM((TAIL, D), jnp.float32),
            pltpu.VMEM((ZROWS, D), jnp.float32),
            pltpu.SemaphoreType.DMA,
        ],
    )(features, ei0.reshape(2 * E), ei1.reshape(2 * E), ei2.reshape(2 * E))


BN = 1000


def _tc_body(p_ref, x_ref, w_ref, o_ref):
    a0 = p_ref[0] + p_ref[3]
    a1 = p_ref[1] + p_ref[4]
    a2 = p_ref[2] + p_ref[5]
    acc = jnp.dot(a0, w_ref[0], preferred_element_type=jnp.float32)
    acc = acc + jnp.dot(a1, w_ref[1], preferred_element_type=jnp.float32)
    acc = acc - jnp.dot(a2, w_ref[2], preferred_element_type=jnp.float32)
    acc = acc + jnp.dot(x_ref[...], w_ref[3], preferred_element_type=jnp.float32)
    o_ref[...] = jnp.maximum(acc, 0.0)


def _tc_finish(partials, features, wstk):
    return pl.pallas_call(
        _tc_body,
        out_shape=jax.ShapeDtypeStruct((N, D), jnp.float32),
        grid=(N // BN,),
        in_specs=[
            pl.BlockSpec((6, BN, D), lambda i: (0, i, 0)),
            pl.BlockSpec((BN, D), lambda i: (i, 0)),
            pl.BlockSpec((4, D, D), lambda i: (0, 0, 0)),
        ],
        out_specs=pl.BlockSpec((BN, D), lambda i: (i, 0)),
    )(partials, features, wstk)


def kernel(features, edge_index_rel0, edge_index_rel1, edge_index_rel2,
           W_rel0, W_rel1, W_rel2, W_self):
    partials = _sc_accumulate(features, edge_index_rel0, edge_index_rel1,
                              edge_index_rel2)
    wstk = jnp.stack([W_rel0, W_rel1, W_rel2, W_self])
    return _tc_finish(partials, features, wstk)
